# baseline (speedup 1.0000x reference)
import functools

import jax
import jax.numpy as jnp
from jax import lax
from jax.experimental import pallas as pl
from jax.experimental.pallas import tpu as pltpu
from jax.experimental.pallas import tpu_sc as plsc

N_NODES = 10000
N_EDGES = 160000
D_IN = 256
HID = 64
HEADS1 = 8
CLASSES = 128

NPAD = 10240
E_TOT = N_EDGES + N_NODES
E_PAD = 172032
MBLK = 512
NBLOCKS = NPAD // MBLK

CHUNK = 64
GT = E_PAD // CHUNK
NCH = GT // 16
NCHW = GT // 32
RW = 144
NACC = 10016
NRT = NACC // 16

NEG = -1e30


def _mm1_body(x_ref, w_ref, s_ref, xwp_ref, a1_ref):
    xw = jnp.dot(x_ref[...], w_ref[...], preferred_element_type=jnp.float32)
    for p in range(HEADS1 // 2):
        xwp_ref[p] = xw[:, p * 2 * HID:(p + 1) * 2 * HID]
    a1_ref[...] = jnp.dot(xw, s_ref[...], preferred_element_type=jnp.float32)


def _mm1(x_pad, W1, S1):
    return pl.pallas_call(
        _mm1_body,
        grid=(NBLOCKS,),
        in_specs=[
            pl.BlockSpec((MBLK, D_IN), lambda i: (i, 0)),
            pl.BlockSpec((D_IN, HEADS1 * HID), lambda i: (0, 0)),
            pl.BlockSpec((HEADS1 * HID, 2 * HEADS1), lambda i: (0, 0)),
        ],
        out_specs=[
            pl.BlockSpec((HEADS1 // 2, MBLK, 2 * HID), lambda i: (0, i, 0)),
            pl.BlockSpec((MBLK, 2 * HEADS1), lambda i: (i, 0)),
        ],
        out_shape=[
            jax.ShapeDtypeStruct((HEADS1 // 2, NPAD, 2 * HID), jnp.float32),
            jax.ShapeDtypeStruct((NPAD, 2 * HEADS1), jnp.float32),
        ],
    )(x_pad, W1, S1)


def _mm2_body(acc_ref, b1_ref, w2_ref, s2_ref, h2w_ref, a2_ref):
    parts = []
    for h in range(HEADS1):
        p, par = divmod(h, 2)
        num = acc_ref[p, :, par * HID:(par + 1) * HID]
        den = acc_ref[p, :, 2 * HID + par:2 * HID + par + 1]
        den = jnp.where(den == 0.0, 1.0, den)
        v = num / den + b1_ref[:, h * HID:(h + 1) * HID]
        parts.append(jnp.where(v > 0, v, jnp.exp(v) - 1.0))
    hmat = jnp.concatenate(parts, axis=1)
    h2w = jnp.dot(hmat, w2_ref[...], preferred_element_type=jnp.float32)
    h2w_ref[...] = h2w
    a2_ref[...] = jnp.dot(h2w, s2_ref[...], preferred_element_type=jnp.float32)


def _mm2(acc1, b1_2d, W2, S2):
    return pl.pallas_call(
        _mm2_body,
        grid=(NBLOCKS,),
        in_specs=[
            pl.BlockSpec((HEADS1 // 2, MBLK, RW), lambda i: (0, i, 0)),
            pl.BlockSpec((1, HEADS1 * HID), lambda i: (0, 0)),
            pl.BlockSpec((HEADS1 * HID, CLASSES), lambda i: (0, 0)),
            pl.BlockSpec((CLASSES, 2), lambda i: (0, 0)),
        ],
        out_specs=[
            pl.BlockSpec((MBLK, CLASSES), lambda i: (i, 0)),
            pl.BlockSpec((MBLK, 2), lambda i: (i, 0)),
        ],
        out_shape=[
            jax.ShapeDtypeStruct((NPAD, CLASSES), jnp.float32),
            jax.ShapeDtypeStruct((NPAD, 2), jnp.float32),
        ],
    )(acc1, b1_2d, W2, S2)


def _pool_body(acc_ref, b2_ref, out_ref, sum_ref):
    i = pl.program_id(0)

    @pl.when(i == 0)
    def _init():
        sum_ref[...] = jnp.zeros_like(sum_ref)

    num = acc_ref[0, :, 0:CLASSES] + acc_ref[1, :, 0:CLASSES]
    den = acc_ref[0, :, CLASSES:CLASSES + 1] + acc_ref[1, :, CLASSES:CLASSES + 1]
    den = jnp.where(den == 0.0, 1.0, den)
    vals = num / den
    rows = i * MBLK + lax.broadcasted_iota(jnp.int32, (MBLK, 1), 0)
    vals = jnp.where(rows < N_NODES, vals, 0.0)
    sum_ref[...] += jnp.sum(vals, axis=0, keepdims=True)

    @pl.when(i == NBLOCKS - 1)
    def _fin():
        t = sum_ref[...] / float(N_NODES) + b2_ref[...]
        m = jnp.max(t)
        e = jnp.exp(t - m)
        out_ref[...] = e / jnp.sum(e)


def _pool(acc2, b2_2d):
    return pl.pallas_call(
        _pool_body,
        grid=(NBLOCKS,),
        in_specs=[
            pl.BlockSpec((2, MBLK, RW), lambda i: (0, i, 0)),
            pl.BlockSpec((1, CLASSES), lambda i: (0, 0)),
        ],
        out_specs=pl.BlockSpec((1, CLASSES), lambda i: (0, 0)),
        out_shape=jax.ShapeDtypeStruct((1, CLASSES), jnp.float32),
        scratch_shapes=[pltpu.VMEM((1, CLASSES), jnp.float32)],
    )(acc2, b2_2d)


_SC_MESH = plsc.VectorSubcoreMesh(
    core_axis_name="c", subcore_axis_name="s", num_cores=2, num_subcores=16)
_SC_PARAMS = pltpu.CompilerParams(
    needs_layout_passes=False, use_tc_tiling_on_sc=False)


def _row_max(vec_ref, hh, tmp_ref):
    def mx(i, carry):
        return jnp.maximum(carry, vec_ref[hh, pl.ds(i * 16, 16)])
    m = lax.fori_loop(0, NPAD // 16, mx, jnp.full((16,), NEG, jnp.float32))
    for k in (1, 2, 4, 8):
        tmp_ref[pl.ds(0, 16)] = m
        idx = lax.iota(jnp.int32, 16) ^ k
        m = jnp.maximum(m, plsc.load_gather(tmp_ref, [idx]))
    return m


def _zero_rows(z_ref, ncol16):
    def zrow(i, _):
        for cc in range(ncol16):
            z_ref[i, pl.ds(cc * 16, 16)] = jnp.zeros((16,), jnp.float32)
        return 0
    lax.fori_loop(0, CHUNK, zrow, 0)


def _sc_p1(asrc1T, adst1T, sd2):
    @functools.partial(
        pl.kernel,
        out_type=jax.ShapeDtypeStruct((HEADS1 // 2, GT, 2, CHUNK), jnp.float32),
        mesh=_SC_MESH,
        compiler_params=_SC_PARAMS,
        scratch_types=dict(
            asrc_v=pltpu.VMEM((4, NPAD), jnp.float32),
            adst_v=pltpu.VMEM((4, NPAD), jnp.float32),
            sd_all=pltpu.VMEM((NCHW, CHUNK), jnp.int32),
            pout_v=pltpu.VMEM((2, NCHW, 2, CHUNK), jnp.float32),
            tmp_v=pltpu.VMEM((16,), jnp.float32),
        ),
    )
    def body(asrc_hbm, adst_hbm, sd_hbm, p1_hbm,
             asrc_v, adst_v, sd_all, pout_v, tmp_v):
        c = lax.axis_index("c")
        s = lax.axis_index("s")
        base = (c * 16 + s) * NCHW
        pltpu.sync_copy(sd_hbm.at[pl.ds(base, NCHW)], sd_all)
        for sp in range(2):
            pltpu.sync_copy(asrc_hbm.at[pl.ds(sp * 4, 4)], asrc_v)
            pltpu.sync_copy(adst_hbm.at[pl.ds(sp * 4, 4)], adst_v)
            bounds = []
            for hh in range(4):
                amax = _row_max(asrc_v, hh, tmp_v) + _row_max(adst_v, hh, tmp_v)
                bounds.append(jnp.where(amax >= 0, amax, 0.2 * amax))

            def chunk(t, carry):
                for q in range(CHUNK // 16):
                    sd = sd_all[t, pl.ds(q * 16, 16)]
                    si = sd & 0xFFFF
                    di = lax.shift_right_logical(sd, 16)
                    for hh in range(4):
                        hv = jnp.full((16,), hh, jnp.int32)
                        sv = (plsc.load_gather(asrc_v, [hv, si])
                              + plsc.load_gather(adst_v, [hv, di]))
                        sv = jnp.where(sv >= 0, sv, 0.2 * sv)
                        pv = jnp.exp(sv - bounds[hh])
                        pout_v[hh // 2, t, hh % 2, pl.ds(q * 16, 16)] = pv
                return carry

            lax.fori_loop(0, NCHW, chunk, 0)
            for k in range(2):
                pltpu.sync_copy(pout_v.at[k],
                                p1_hbm.at[sp * 2 + k, pl.ds(base, NCHW)])

    return body(asrc1T, adst1T, sd2)


def _sc_p2(asrc2T, adst2T, sd2):
    @functools.partial(
        pl.kernel,
        out_type=jax.ShapeDtypeStruct((GT, CHUNK), jnp.float32),
        mesh=_SC_MESH,
        compiler_params=_SC_PARAMS,
        scratch_types=dict(
            asrc_v=pltpu.VMEM((1, NPAD), jnp.float32),
            adst_v=pltpu.VMEM((1, NPAD), jnp.float32),
            sd_all=pltpu.VMEM((NCHW, CHUNK), jnp.int32),
            pout_v=pltpu.VMEM((NCHW, CHUNK), jnp.float32),
            tmp_v=pltpu.VMEM((16,), jnp.float32),
        ),
    )
    def body(asrc_hbm, adst_hbm, sd_hbm, p2_hbm,
             asrc_v, adst_v, sd_all, pout_v, tmp_v):
        c = lax.axis_index("c")
        s = lax.axis_index("s")
        base = (c * 16 + s) * NCHW
        pltpu.sync_copy(sd_hbm.at[pl.ds(base, NCHW)], sd_all)
        pltpu.sync_copy(asrc_hbm, asrc_v)
        pltpu.sync_copy(adst_hbm, adst_v)
        amax = _row_max(asrc_v, 0, tmp_v) + _row_max(adst_v, 0, tmp_v)
        bound = jnp.where(amax >= 0, amax, 0.2 * amax)

        def chunk(t, carry):
            for q in range(CHUNK // 16):
                sd = sd_all[t, pl.ds(q * 16, 16)]
                si = sd & 0xFFFF
                di = lax.shift_right_logical(sd, 16)
                zv = jnp.zeros((16,), jnp.int32)
                sv = (plsc.load_gather(asrc_v, [zv, si])
                      + plsc.load_gather(adst_v, [zv, di]))
                sv = jnp.where(sv >= 0, sv, 0.2 * sv)
                pout_v[t, pl.ds(q * 16, 16)] = jnp.exp(sv - bound)
            return carry

        lax.fori_loop(0, NCHW, chunk, 0)
        pltpu.sync_copy(pout_v, p2_hbm.at[pl.ds(base, NCHW)])

    return body(asrc2T, adst2T, sd2)


_HEAVY_KEYS = ('acc_sp', 'sd0_v', 'sd1_v', 'si0_v', 'si1_v', 'di0_v', 'di1_v',
               'g0_v', 'g1_v', 'rows0_v', 'rows1_v', 'pb0_v', 'pb1_v',
               'dsem0', 'dsem1', 'gsem0', 'gsem1', 'ssem0', 'ssem1',
               'psem0', 'psem1')
_HEAVY_SCRATCH = dict(
    acc_sp=pltpu.VMEM_SHARED((NACC, RW), jnp.float32),
    sd0_v=pltpu.VMEM((CHUNK,), jnp.int32),
    sd1_v=pltpu.VMEM((CHUNK,), jnp.int32),
    si0_v=pltpu.VMEM((CHUNK,), jnp.int32),
    si1_v=pltpu.VMEM((CHUNK,), jnp.int32),
    di0_v=pltpu.VMEM((CHUNK,), jnp.int32),
    di1_v=pltpu.VMEM((CHUNK,), jnp.int32),
    g0_v=pltpu.VMEM((CHUNK, CLASSES), jnp.float32),
    g1_v=pltpu.VMEM((CHUNK, CLASSES), jnp.float32),
    rows0_v=pltpu.VMEM((CHUNK, RW), jnp.float32),
    rows1_v=pltpu.VMEM((CHUNK, RW), jnp.float32),
    pb0_v=pltpu.VMEM((2, CHUNK), jnp.float32),
    pb1_v=pltpu.VMEM((2, CHUNK), jnp.float32),
    dsem0=pltpu.SemaphoreType.DMA,
    dsem1=pltpu.SemaphoreType.DMA,
    gsem0=pltpu.SemaphoreType.DMA,
    gsem1=pltpu.SemaphoreType.DMA,
    ssem0=pltpu.SemaphoreType.DMA,
    ssem1=pltpu.SemaphoreType.DMA,
    psem0=pltpu.SemaphoreType.DMA,
    psem1=pltpu.SemaphoreType.DMA,
)


def _heavy_pass(tbl_hbm, sd_row, p_row, npair, nchunks, scr):
    acc_sp = scr['acc_sp']
    bufs = []
    for b in range(2):
        pb_full = scr[f'pb{b}_v']
        pb_dst = pb_full if npair == 2 else pb_full.at[pl.ds(0, 1)]
        bufs.append((scr[f'sd{b}_v'], scr[f'si{b}_v'], scr[f'di{b}_v'],
                     scr[f'g{b}_v'], scr[f'rows{b}_v'], pb_full, pb_dst,
                     scr[f'dsem{b}'], scr[f'gsem{b}'], scr[f'ssem{b}'],
                     scr[f'psem{b}']))

    for b in range(2):
        sd_v, si_v, di_v, g_v, rows_v, pb_v, pb_dst, dsem, gsem, ssem, psem = bufs[b]
        pltpu.async_copy(sd_row(b), sd_v, dsem)
        pltpu.async_copy(p_row(b), pb_dst, psem)

    def step(i, carry):
        for b in range(2):
            t = i * 2 + b
            sd_v, si_v, di_v, g_v, rows_v, pb_v, pb_dst, dsem, gsem, ssem, psem = bufs[b]
            pltpu.make_async_copy(sd_row(0), sd_v, dsem).wait()
            for q in range(CHUNK // 16):
                sd = sd_v[pl.ds(q * 16, 16)]
                si_v[pl.ds(q * 16, 16)] = sd & 0xFFFF
                di_v[pl.ds(q * 16, 16)] = lax.shift_right_logical(sd, 16)
            pltpu.async_copy(tbl_hbm.at[si_v], g_v, gsem)
            pltpu.make_async_copy(p_row(0), pb_dst, psem).wait()

            @pl.when(t >= 2)
            def _drain():
                pltpu.make_async_copy(rows_v, acc_sp.at[di_v], ssem).wait()

            for q in range(CHUNK // 16):
                rowi = lax.iota(jnp.int32, 16) + q * 16
                for par in range(npair):
                    plsc.store_scatter(
                        rows_v, [rowi, jnp.full((16,), CLASSES + par, jnp.int32)],
                        pb_v[par, pl.ds(q * 16, 16)])
            pltpu.make_async_copy(tbl_hbm.at[si_v], g_v, gsem).wait()
            ncol = (CLASSES // 16) // npair
            for j in range(CHUNK):
                jv = jnp.full((16,), j, jnp.int32)
                for par in range(npair):
                    pj = plsc.load_gather(
                        pb_v, [jnp.full((16,), par, jnp.int32), jv])
                    for cc in range(par * ncol, (par + 1) * ncol):
                        rows_v[j, pl.ds(cc * 16, 16)] = \
                            g_v[j, pl.ds(cc * 16, 16)] * pj
            pltpu.async_copy(rows_v, acc_sp.at[di_v], ssem, add=True)

            @pl.when(t + 2 < nchunks)
            def _next():
                pltpu.async_copy(sd_row(t + 2), sd_v, dsem)
                pltpu.async_copy(p_row(t + 2), pb_dst, psem)
        return carry

    lax.fori_loop(0, nchunks // 2, step, 0)
    for b in range(2):
        sd_v, si_v, di_v, g_v, rows_v, pb_v, pb_dst, dsem, gsem, ssem, psem = bufs[b]
        pltpu.make_async_copy(rows_v, acc_sp.at[di_v], ssem).wait()


def _zero_acc(scr, s):
    rows0_v = scr['rows0_v']
    acc_sp = scr['acc_sp']
    _zero_rows(rows0_v, RW // 16)
    _zero_rows(scr['rows1_v'], RW // 16)
    for q in range(NRT // CHUNK):
        pltpu.sync_copy(rows0_v, acc_sp.at[pl.ds(s * NRT + q * CHUNK, CHUNK)])
    pltpu.sync_copy(rows0_v.at[pl.ds(0, NRT % CHUNK)],
                    acc_sp.at[pl.ds(s * NRT + (NRT // CHUNK) * CHUNK,
                                    NRT % CHUNK)])


def _sc_edge1(xwp, p1, sd2):
    @functools.partial(
        pl.kernel,
        out_type=jax.ShapeDtypeStruct((HEADS1 // 2, NPAD, RW), jnp.float32),
        mesh=_SC_MESH,
        compiler_params=_SC_PARAMS,
        scratch_types=_HEAVY_SCRATCH,
    )
    def body(xwp_hbm, p1_hbm, sd_hbm, out_hbm, **scr):
        c = lax.axis_index("c")
        s = lax.axis_index("s")
        for pp in range(2):
            pr = c * 2 + pp
            _zero_acc(scr, s)
            plsc.subcore_barrier()
            _heavy_pass(xwp_hbm.at[pr],
                        lambda t: sd_hbm.at[s * NCH + t],
                        lambda t: p1_hbm.at[pr, s * NCH + t],
                        2, NCH, scr)
            plsc.subcore_barrier()
            pltpu.sync_copy(scr['acc_sp'].at[pl.ds(s * NRT, NRT)],
                            out_hbm.at[pr, pl.ds(s * NRT, NRT)])
            plsc.subcore_barrier()

    return body(xwp, p1, sd2)


def _sc_edge2(h2w, p2, sd2):
    @functools.partial(
        pl.kernel,
        out_type=jax.ShapeDtypeStruct((2, NPAD, RW), jnp.float32),
        mesh=_SC_MESH,
        compiler_params=_SC_PARAMS,
        scratch_types=_HEAVY_SCRATCH,
    )
    def body(h2w_hbm, p2_hbm, sd_hbm, out_hbm, **scr):
        c = lax.axis_index("c")
        s = lax.axis_index("s")
        base = (c * 16 + s) * NCHW
        _zero_acc(scr, s)
        plsc.subcore_barrier()
        _heavy_pass(h2w_hbm,
                    lambda t: sd_hbm.at[base + t],
                    lambda t: p2_hbm.at[pl.ds(base + t, 1)],
                    1, NCHW, scr)
        plsc.subcore_barrier()
        pltpu.sync_copy(scr['acc_sp'].at[pl.ds(s * NRT, NRT)],
                        out_hbm.at[c, pl.ds(s * NRT, NRT)])

    return body(h2w, p2, sd2)


def kernel(x, edge_index, W1, att_src1, att_dst1, b1, W2, att_src2, att_dst2, b2):
    f32 = jnp.float32
    loop = jnp.arange(N_NODES, dtype=jnp.int32)
    pad = N_NODES + (jnp.arange(E_PAD - E_TOT, dtype=jnp.int32) % 16)
    src = jnp.concatenate([edge_index[0].astype(jnp.int32), loop, pad])
    dst = jnp.concatenate([edge_index[1].astype(jnp.int32), loop, pad])
    sd2 = (src | (dst << 16)).reshape(GT, CHUNK)
    x_pad = jnp.concatenate([x, jnp.zeros((NPAD - N_NODES, D_IN), f32)], axis=0)

    eye1 = jnp.eye(HEADS1, dtype=f32)[:, None, :]
    S1s = (att_src1[0][:, :, None] * eye1).reshape(HEADS1 * HID, HEADS1)
    S1d = (att_dst1[0][:, :, None] * eye1).reshape(HEADS1 * HID, HEADS1)
    S1 = jnp.concatenate([S1s, S1d], axis=1)
    S2 = jnp.concatenate([att_src2[0].T, att_dst2[0].T], axis=1)

    xwp, a1 = _mm1(x_pad, W1, S1)
    a1T = jnp.concatenate(
        [a1[:N_NODES].T.reshape(2 * HEADS1, N_NODES),
         jnp.full((2 * HEADS1, NPAD - N_NODES), NEG, f32)], axis=1)
    asrc1T, adst1T = a1T[:HEADS1], a1T[HEADS1:]

    p1 = _sc_p1(asrc1T, adst1T, sd2)
    acc1 = _sc_edge1(xwp, p1, sd2)

    h2w, a2 = _mm2(acc1, b1.reshape(1, -1), W2, S2)
    a2T = jnp.concatenate(
        [a2[:N_NODES].T.reshape(2, N_NODES),
         jnp.full((2, NPAD - N_NODES), NEG, f32)], axis=1)

    p2 = _sc_p2(a2T[0:1], a2T[1:2], sd2)
    acc2 = _sc_edge2(h2w, p2, sd2)

    return _pool(acc2, b2.reshape(1, -1))


# speedup vs baseline: 1.5104x; 1.5104x over previous
import functools

import jax
import jax.numpy as jnp
from jax import lax
from jax.experimental import pallas as pl
from jax.experimental.pallas import tpu as pltpu
from jax.experimental.pallas import tpu_sc as plsc

N_NODES = 10000
N_EDGES = 160000
D_IN = 256
HID = 64
HEADS1 = 8
CLASSES = 128

NPAD = 10240
PAD_IDX = 10008
E_TOT = N_EDGES + N_NODES
E_PAD = 172032
MBLK = 512
NBLOCKS = NPAD // MBLK

CHUNK = 64
NCH = E_PAD // 16 // CHUNK
FD = HID + 16
NACC = 10016
NRT = NACC // 16

NEG = -1e30


def _mm1_body(x_ref, w_ref, s_ref, xwt_ref, a1_ref):
    xw = jnp.dot(x_ref[...], w_ref[...], preferred_element_type=jnp.float32)
    for h in range(HEADS1):
        xwt_ref[h] = xw[:, h * HID:(h + 1) * HID]
    a1_ref[...] = jnp.dot(xw, s_ref[...], preferred_element_type=jnp.float32)


def _mm1(x_pad, W1, S1):
    return pl.pallas_call(
        _mm1_body,
        grid=(NBLOCKS,),
        in_specs=[
            pl.BlockSpec((MBLK, D_IN), lambda i: (i, 0)),
            pl.BlockSpec((D_IN, HEADS1 * HID), lambda i: (0, 0)),
            pl.BlockSpec((HEADS1 * HID, 2 * HEADS1), lambda i: (0, 0)),
        ],
        out_specs=[
            pl.BlockSpec((HEADS1, MBLK, HID), lambda i: (0, i, 0)),
            pl.BlockSpec((MBLK, 2 * HEADS1), lambda i: (i, 0)),
        ],
        out_shape=[
            jax.ShapeDtypeStruct((HEADS1, NPAD, HID), jnp.float32),
            jax.ShapeDtypeStruct((NPAD, 2 * HEADS1), jnp.float32),
        ],
    )(x_pad, W1, S1)


def _mm2_body(acc_ref, b1_ref, w2_ref, s2_ref, h2w_ref, a2_ref):
    parts = []
    for h in range(HEADS1):
        num = acc_ref[h, :, 0:HID]
        den = acc_ref[h, :, HID:HID + 1]
        den = jnp.where(den == 0.0, 1.0, den)
        v = num / den + b1_ref[:, h * HID:(h + 1) * HID]
        parts.append(jnp.where(v > 0, v, jnp.exp(v) - 1.0))
    hmat = jnp.concatenate(parts, axis=1)
    h2w = jnp.dot(hmat, w2_ref[...], preferred_element_type=jnp.float32)
    h2w_ref[0] = h2w[:, 0:HID]
    h2w_ref[1] = h2w[:, HID:2 * HID]
    a2_ref[...] = jnp.dot(h2w, s2_ref[...], preferred_element_type=jnp.float32)


def _mm2(acc1, b1_2d, W2, S2):
    return pl.pallas_call(
        _mm2_body,
        grid=(NBLOCKS,),
        in_specs=[
            pl.BlockSpec((HEADS1, MBLK, FD), lambda i: (0, i, 0)),
            pl.BlockSpec((1, HEADS1 * HID), lambda i: (0, 0)),
            pl.BlockSpec((HEADS1 * HID, CLASSES), lambda i: (0, 0)),
            pl.BlockSpec((CLASSES, 2), lambda i: (0, 0)),
        ],
        out_specs=[
            pl.BlockSpec((2, MBLK, HID), lambda i: (0, i, 0)),
            pl.BlockSpec((MBLK, 2), lambda i: (i, 0)),
        ],
        out_shape=[
            jax.ShapeDtypeStruct((2, NPAD, HID), jnp.float32),
            jax.ShapeDtypeStruct((NPAD, 2), jnp.float32),
        ],
    )(acc1, b1_2d, W2, S2)


def _pool_body(acc_ref, b2_ref, out_ref, sum_ref):
    i = pl.program_id(0)

    @pl.when(i == 0)
    def _init():
        sum_ref[...] = jnp.zeros_like(sum_ref)

    num = jnp.concatenate([acc_ref[0, :, 0:HID], acc_ref[1, :, 0:HID]], axis=1)
    den = acc_ref[0, :, HID:HID + 1]
    den = jnp.where(den == 0.0, 1.0, den)
    vals = num / den
    rows = i * MBLK + lax.broadcasted_iota(jnp.int32, (MBLK, 1), 0)
    vals = jnp.where(rows < N_NODES, vals, 0.0)
    sum_ref[...] += jnp.sum(vals, axis=0, keepdims=True)

    @pl.when(i == NBLOCKS - 1)
    def _fin():
        t = sum_ref[...] / float(N_NODES) + b2_ref[...]
        m = jnp.max(t)
        e = jnp.exp(t - m)
        out_ref[...] = e / jnp.sum(e)


def _pool(acc2, b2_2d):
    return pl.pallas_call(
        _pool_body,
        grid=(NBLOCKS,),
        in_specs=[
            pl.BlockSpec((2, MBLK, FD), lambda i: (0, i, 0)),
            pl.BlockSpec((1, CLASSES), lambda i: (0, 0)),
        ],
        out_specs=pl.BlockSpec((1, CLASSES), lambda i: (0, 0)),
        out_shape=jax.ShapeDtypeStruct((1, CLASSES), jnp.float32),
        scratch_shapes=[pltpu.VMEM((1, CLASSES), jnp.float32)],
    )(acc2, b2_2d)


_SC_MESH = plsc.VectorSubcoreMesh(
    core_axis_name="c", subcore_axis_name="s", num_cores=2, num_subcores=16)
_SC_PARAMS = pltpu.CompilerParams(
    needs_layout_passes=False, use_tc_tiling_on_sc=False)


def _tile_max(vec_ref, n16, tmp_ref):
    def mx(i, carry):
        return jnp.maximum(carry, vec_ref[pl.ds(i * 16, 16)])
    m = lax.fori_loop(0, n16, mx, jnp.full((16,), NEG, jnp.float32))
    for k in (1, 2, 4, 8):
        tmp_ref[pl.ds(0, 16)] = m
        idx = lax.iota(jnp.int32, 16) ^ k
        m = jnp.maximum(m, plsc.load_gather(tmp_ref, [idx]))
    return m


def _zero_rows(z_ref):
    def zrow(i, _):
        for cc in range(FD // 16):
            z_ref[i, pl.ds(cc * 16, 16)] = jnp.zeros((16,), jnp.float32)
        return 0
    lax.fori_loop(0, CHUNK, zrow, 0)


def _edge_pass(tbl_sp, asrc_v, adst_v, sd_hbm, sd0_v, sd1_v,
               si0_v, si1_v, di0_v, di1_v, g0_v, g1_v, rows0_v, rows1_v,
               p_v, acc_sp, dsem0, dsem1, sem0, sem1, ssem0, ssem1):
    amax = _tile_max(asrc_v, NPAD // 16, p_v) + _tile_max(adst_v, NPAD // 16, p_v)
    bound = jnp.where(amax >= 0, amax, 0.2 * amax)
    bufs = ((sd0_v, si0_v, di0_v, g0_v, rows0_v, dsem0, sem0, ssem0),
            (sd1_v, si1_v, di1_v, g1_v, rows1_v, dsem1, sem1, ssem1))

    pltpu.async_copy(sd_hbm.at[0], sd0_v, dsem0)
    pltpu.async_copy(sd_hbm.at[1], sd1_v, dsem1)

    def step(i, carry):
        for b in range(2):
            t = i * 2 + b
            sd_v, si_v, di_v, g_v, rows_v, dsem, sem, ssem = bufs[b]
            pltpu.make_async_copy(sd_hbm.at[0], sd_v, dsem).wait()
            for q in range(CHUNK // 16):
                sd = sd_v[pl.ds(q * 16, 16)]
                si_v[pl.ds(q * 16, 16)] = sd & 0xFFFF
                di_v[pl.ds(q * 16, 16)] = lax.shift_right_logical(sd, 16)
            pltpu.async_copy(tbl_sp.at[si_v], g_v, sem)
            for q in range(CHUNK // 16):
                sd = sd_v[pl.ds(q * 16, 16)]
                si = sd & 0xFFFF
                di = lax.shift_right_logical(sd, 16)
                sv = plsc.load_gather(asrc_v, [si]) + plsc.load_gather(adst_v, [di])
                sv = jnp.where(sv >= 0, sv, 0.2 * sv)
                pv = jnp.exp(sv - bound)
                p_v[pl.ds(q * 16, 16)] = pv

            @pl.when(t >= 2)
            def _drain():
                pltpu.make_async_copy(rows_v, acc_sp.at[di0_v], ssem).wait()

            for q in range(CHUNK // 16):
                rowi = lax.iota(jnp.int32, 16) + q * 16
                plsc.store_scatter(rows_v, [rowi, jnp.full((16,), HID, jnp.int32)],
                                   p_v[pl.ds(q * 16, 16)])
            pltpu.make_async_copy(tbl_sp.at[si0_v], g_v, sem).wait()
            for j in range(CHUNK):
                pj = plsc.load_gather(p_v, [jnp.full((16,), j, jnp.int32)])
                for cc in range(HID // 16):
                    rows_v[j, pl.ds(cc * 16, 16)] = g_v[j, pl.ds(cc * 16, 16)] * pj
            pltpu.async_copy(rows_v, acc_sp.at[di_v], ssem, add=True)

            @pl.when(t + 2 < NCH)
            def _next():
                pltpu.async_copy(sd_hbm.at[t + 2], sd_v, dsem)
        return carry

    lax.fori_loop(0, NCH // 2, step, 0)
    pltpu.make_async_copy(rows0_v, acc_sp.at[di0_v], ssem0).wait()
    pltpu.make_async_copy(rows1_v, acc_sp.at[di1_v], ssem1).wait()


_SC_SCRATCH = dict(
    acc_sp=pltpu.VMEM_SHARED((NACC, FD), jnp.float32),
    tbl_sp=pltpu.VMEM_SHARED((NACC, HID), jnp.float32),
    asrc_v=pltpu.VMEM((NPAD,), jnp.float32),
    adst_v=pltpu.VMEM((NPAD,), jnp.float32),
    sd0_v=pltpu.VMEM((CHUNK,), jnp.int32),
    sd1_v=pltpu.VMEM((CHUNK,), jnp.int32),
    si0_v=pltpu.VMEM((CHUNK,), jnp.int32),
    si1_v=pltpu.VMEM((CHUNK,), jnp.int32),
    di0_v=pltpu.VMEM((CHUNK,), jnp.int32),
    di1_v=pltpu.VMEM((CHUNK,), jnp.int32),
    g0_v=pltpu.VMEM((CHUNK, HID), jnp.float32),
    g1_v=pltpu.VMEM((CHUNK, HID), jnp.float32),
    rows0_v=pltpu.VMEM((CHUNK, FD), jnp.float32),
    rows1_v=pltpu.VMEM((CHUNK, FD), jnp.float32),
    p_v=pltpu.VMEM((CHUNK,), jnp.float32),
    dsem0=pltpu.SemaphoreType.DMA,
    dsem1=pltpu.SemaphoreType.DMA,
    sem0=pltpu.SemaphoreType.DMA,
    sem1=pltpu.SemaphoreType.DMA,
    ssem0=pltpu.SemaphoreType.DMA,
    ssem1=pltpu.SemaphoreType.DMA,
)


def _sc_edge1(xwt, asrc1T, adst1T, sd3):
    @functools.partial(
        pl.kernel,
        out_type=jax.ShapeDtypeStruct((HEADS1, NPAD, FD), jnp.float32),
        mesh=_SC_MESH,
        compiler_params=_SC_PARAMS,
        scratch_types=_SC_SCRATCH,
    )
    def body(xwt_hbm, asrc_hbm, adst_hbm, sd_hbm, out_hbm,
             acc_sp, tbl_sp, asrc_v, adst_v, sd0_v, sd1_v, si0_v, si1_v,
             di0_v, di1_v, g0_v, g1_v, rows0_v, rows1_v, p_v,
             dsem0, dsem1, sem0, sem1, ssem0, ssem1):
        c = lax.axis_index("c")
        s = lax.axis_index("s")
        for hh in range(HEADS1 // 2):
            h = c * (HEADS1 // 2) + hh
            _zero_rows(rows0_v)
            _zero_rows(rows1_v)
            pltpu.sync_copy(asrc_hbm.at[h], asrc_v)
            pltpu.sync_copy(adst_hbm.at[h], adst_v)
            pltpu.sync_copy(xwt_hbm.at[h, pl.ds(s * NRT, NRT)],
                            tbl_sp.at[pl.ds(s * NRT, NRT)])
            for q in range(NRT // CHUNK):
                pltpu.sync_copy(rows0_v, acc_sp.at[pl.ds(s * NRT + q * CHUNK, CHUNK)])
            pltpu.sync_copy(rows0_v.at[pl.ds(0, NRT % CHUNK)],
                            acc_sp.at[pl.ds(s * NRT + (NRT // CHUNK) * CHUNK,
                                            NRT % CHUNK)])
            plsc.subcore_barrier()
            _edge_pass(tbl_sp, asrc_v, adst_v, sd_hbm.at[s],
                       sd0_v, sd1_v, si0_v, si1_v, di0_v, di1_v,
                       g0_v, g1_v, rows0_v, rows1_v, p_v, acc_sp,
                       dsem0, dsem1, sem0, sem1, ssem0, ssem1)
            plsc.subcore_barrier()
            pltpu.sync_copy(acc_sp.at[pl.ds(s * NRT, NRT)],
                            out_hbm.at[h, pl.ds(s * NRT, NRT)])
            plsc.subcore_barrier()

    return body(xwt, asrc1T, adst1T, sd3)


def _sc_edge2(h2w3, asrc2T, adst2T, sd3):
    @functools.partial(
        pl.kernel,
        out_type=jax.ShapeDtypeStruct((2, NPAD, FD), jnp.float32),
        mesh=_SC_MESH,
        compiler_params=_SC_PARAMS,
        scratch_types=_SC_SCRATCH,
    )
    def body(h2w_hbm, asrc_hbm, adst_hbm, sd_hbm, out_hbm,
             acc_sp, tbl_sp, asrc_v, adst_v, sd0_v, sd1_v, si0_v, si1_v,
             di0_v, di1_v, g0_v, g1_v, rows0_v, rows1_v, p_v,
             dsem0, dsem1, sem0, sem1, ssem0, ssem1):
        c = lax.axis_index("c")
        s = lax.axis_index("s")
        _zero_rows(rows0_v)
        _zero_rows(rows1_v)
        pltpu.sync_copy(asrc_hbm, asrc_v)
        pltpu.sync_copy(adst_hbm, adst_v)
        pltpu.sync_copy(h2w_hbm.at[c, pl.ds(s * NRT, NRT)],
                        tbl_sp.at[pl.ds(s * NRT, NRT)])
        for q in range(NRT // CHUNK):
            pltpu.sync_copy(rows0_v, acc_sp.at[pl.ds(s * NRT + q * CHUNK, CHUNK)])
        pltpu.sync_copy(rows0_v.at[pl.ds(0, NRT % CHUNK)],
                        acc_sp.at[pl.ds(s * NRT + (NRT // CHUNK) * CHUNK,
                                        NRT % CHUNK)])
        plsc.subcore_barrier()
        _edge_pass(tbl_sp, asrc_v, adst_v, sd_hbm.at[s],
                   sd0_v, sd1_v, si0_v, si1_v, di0_v, di1_v,
                   g0_v, g1_v, rows0_v, rows1_v, p_v, acc_sp,
                   dsem0, dsem1, sem0, sem1, ssem0, ssem1)
        plsc.subcore_barrier()
        pltpu.sync_copy(acc_sp.at[pl.ds(s * NRT, NRT)],
                        out_hbm.at[c, pl.ds(s * NRT, NRT)])

    return body(h2w3, asrc2T, adst2T, sd3)


def kernel(x, edge_index, W1, att_src1, att_dst1, b1, W2, att_src2, att_dst2, b2):
    f32 = jnp.float32
    loop = jnp.arange(N_NODES, dtype=jnp.int32)
    pad = N_NODES + (jnp.arange(E_PAD - E_TOT, dtype=jnp.int32) % 16)
    src = jnp.concatenate([edge_index[0].astype(jnp.int32), loop, pad])
    dst = jnp.concatenate([edge_index[1].astype(jnp.int32), loop, pad])
    sd3 = (src | (dst << 16)).reshape(16, NCH, CHUNK)
    x_pad = jnp.concatenate([x, jnp.zeros((NPAD - N_NODES, D_IN), f32)], axis=0)

    eye1 = jnp.eye(HEADS1, dtype=f32)[:, None, :]
    S1s = (att_src1[0][:, :, None] * eye1).reshape(HEADS1 * HID, HEADS1)
    S1d = (att_dst1[0][:, :, None] * eye1).reshape(HEADS1 * HID, HEADS1)
    S1 = jnp.concatenate([S1s, S1d], axis=1)
    S2 = jnp.concatenate([att_src2[0].T, att_dst2[0].T], axis=1)

    xwt, a1 = _mm1(x_pad, W1, S1)
    a1T = jnp.concatenate(
        [a1[:N_NODES].T.reshape(2 * HEADS1, N_NODES),
         jnp.full((2 * HEADS1, NPAD - N_NODES), NEG, f32)], axis=1)
    asrc1T, adst1T = a1T[:HEADS1], a1T[HEADS1:]

    acc1 = _sc_edge1(xwt, asrc1T, adst1T, sd3)

    h2w3, a2 = _mm2(acc1, b1.reshape(1, -1), W2, S2)
    a2T = jnp.concatenate(
        [a2[:N_NODES].T.reshape(2, N_NODES),
         jnp.full((2, NPAD - N_NODES), NEG, f32)], axis=1)

    acc2 = _sc_edge2(h2w3, a2T[0], a2T[1], sd3)

    return _pool(acc2, b2.reshape(1, -1))
